# baseline (speedup 1.0000x reference)
import jax
import jax.numpy as jnp
from jax.experimental import pallas as pl
from jax.experimental.pallas import tpu as pltpu

_ROWS = 4096
_COLS = 128
_GRID = 1
_BR = _ROWS // _GRID
_NIDX = 26
_TVLEN = 1000000


def _body(ti_smem, w_smem, tv_any, f_vmem, out_vmem, scr_smem, m_smem, sem):
    pid = pl.program_id(0)

    @pl.when(pid == 0)
    def _():
        cps = [
            pltpu.make_async_copy(
                tv_any.at[pl.ds(
                    pl.multiple_of((ti_smem[i] // 128) * 128, 128), 128)],
                scr_smem.at[i], sem)
            for i in range(_NIDX)
        ] + [
            pltpu.make_async_copy(
                tv_any.at[pl.ds(
                    pl.multiple_of(
                        (ti_smem[0] * 0) + ((_TVLEN // 128) * 128), 128),
                    128)],
                scr_smem.at[_NIDX], sem)
        ]
        for cp in cps:
            cp.start()
        for cp in cps:
            cp.wait()
        s = scr_smem[0, ti_smem[0] % 128]
        for i in range(1, _NIDX):
            s = s + scr_smem[i, ti_smem[i] % 128]
        m_smem[0] = s * w_smem[0]

    out_vmem[...] = f_vmem[...] * m_smem[0]


@jax.jit
def kernel(f, ti, tv, weight):
    out = pl.pallas_call(
        _body,
        grid=(_GRID,),
        in_specs=[
            pl.BlockSpec(memory_space=pltpu.SMEM),
            pl.BlockSpec(memory_space=pltpu.SMEM),
            pl.BlockSpec(memory_space=pl.ANY),
            pl.BlockSpec((_BR, _COLS), lambda i: (i, 0)),
        ],
        out_specs=pl.BlockSpec((_BR, _COLS), lambda i: (i, 0)),
        out_shape=jax.ShapeDtypeStruct((_ROWS, _COLS), jnp.float32),
        scratch_shapes=[
            pltpu.SMEM((_NIDX + 1, 128), jnp.float32),
            pltpu.SMEM((1,), jnp.float32),
            pltpu.SemaphoreType.DMA,
        ],
        compiler_params=pltpu.CompilerParams(
            dimension_semantics=("arbitrary",)),
    )(ti.astype(jnp.int32), weight, tv, f)
    return out


# speedup vs baseline: 1.1504x; 1.1504x over previous
import jax
import jax.numpy as jnp
from jax.experimental import pallas as pl
from jax.experimental.pallas import tpu as pltpu

_ROWS = 4096
_COLS = 128
_NCH = 4
_CR = _ROWS // _NCH
_NIDX = 26
_TVLEN = 1000000


def _body(ti_smem, w_smem, tv_any, f_any, out_any, fv, scr_smem,
          sem_g, sem_in, sem_out):
    cps_in = [
        pltpu.make_async_copy(
            f_any.at[pl.ds(c * _CR, _CR), :], fv.at[c], sem_in)
        for c in range(_NCH)
    ]
    for cp in cps_in:
        cp.start()

    cps_g = [
        pltpu.make_async_copy(
            tv_any.at[pl.ds(
                pl.multiple_of((ti_smem[i] // 128) * 128, 128), 128)],
            scr_smem.at[i], sem_g)
        for i in range(_NIDX)
    ] + [
        pltpu.make_async_copy(
            tv_any.at[pl.ds(
                pl.multiple_of(
                    (ti_smem[0] * 0) + ((_TVLEN // 128) * 128), 128),
                128)],
            scr_smem.at[_NIDX], sem_g)
    ]
    for cp in cps_g:
        cp.start()
    for cp in cps_g:
        cp.wait()
    s = scr_smem[0, ti_smem[0] % 128]
    for i in range(1, _NIDX):
        s = s + scr_smem[i, ti_smem[i] % 128]
    m = s * w_smem[0]

    cps_out = [
        pltpu.make_async_copy(
            fv.at[c], out_any.at[pl.ds(c * _CR, _CR), :], sem_out)
        for c in range(_NCH)
    ]
    for c in range(_NCH):
        cps_in[c].wait()
        fv[c] = fv[c] * m
        cps_out[c].start()
    for cp in cps_out:
        cp.wait()


@jax.jit
def kernel(f, ti, tv, weight):
    out = pl.pallas_call(
        _body,
        in_specs=[
            pl.BlockSpec(memory_space=pltpu.SMEM),
            pl.BlockSpec(memory_space=pltpu.SMEM),
            pl.BlockSpec(memory_space=pl.ANY),
            pl.BlockSpec(memory_space=pl.ANY),
        ],
        out_specs=pl.BlockSpec(memory_space=pl.ANY),
        out_shape=jax.ShapeDtypeStruct((_ROWS, _COLS), jnp.float32),
        scratch_shapes=[
            pltpu.VMEM((_NCH, _CR, _COLS), jnp.float32),
            pltpu.SMEM((_NIDX + 1, 128), jnp.float32),
            pltpu.SemaphoreType.DMA,
            pltpu.SemaphoreType.DMA,
            pltpu.SemaphoreType.DMA,
        ],
    )(ti.astype(jnp.int32), weight, tv, f)
    return out


# speedup vs baseline: 1.1549x; 1.0039x over previous
import jax
import jax.numpy as jnp
from jax.experimental import pallas as pl
from jax.experimental.pallas import tpu as pltpu

_ROWS = 4096
_COLS = 128
_NCH = 8
_CR = _ROWS // _NCH
_NIDX = 26
_TVLEN = 1000000


def _body(ti_smem, w_smem, tv_any, f_any, out_any, fv, scr_smem,
          sem_g, sem_in, sem_out):
    cps_in = [
        pltpu.make_async_copy(
            f_any.at[pl.ds(c * _CR, _CR), :], fv.at[c], sem_in)
        for c in range(_NCH)
    ]
    for cp in cps_in:
        cp.start()

    cps_g = [
        pltpu.make_async_copy(
            tv_any.at[pl.ds(
                pl.multiple_of((ti_smem[i] // 128) * 128, 128), 128)],
            scr_smem.at[i], sem_g)
        for i in range(_NIDX)
    ] + [
        pltpu.make_async_copy(
            tv_any.at[pl.ds(
                pl.multiple_of(
                    (ti_smem[0] * 0) + ((_TVLEN // 128) * 128), 128),
                128)],
            scr_smem.at[_NIDX], sem_g)
    ]
    for cp in cps_g:
        cp.start()
    for cp in cps_g:
        cp.wait()
    s = scr_smem[0, ti_smem[0] % 128]
    for i in range(1, _NIDX):
        s = s + scr_smem[i, ti_smem[i] % 128]
    m = s * w_smem[0]

    cps_out = [
        pltpu.make_async_copy(
            fv.at[c], out_any.at[pl.ds(c * _CR, _CR), :], sem_out)
        for c in range(_NCH)
    ]
    for c in range(_NCH):
        cps_in[c].wait()
        fv[c] = fv[c] * m
        cps_out[c].start()
    for cp in cps_out:
        cp.wait()


@jax.jit
def kernel(f, ti, tv, weight):
    out = pl.pallas_call(
        _body,
        in_specs=[
            pl.BlockSpec(memory_space=pltpu.SMEM),
            pl.BlockSpec(memory_space=pltpu.SMEM),
            pl.BlockSpec(memory_space=pl.ANY),
            pl.BlockSpec(memory_space=pl.ANY),
        ],
        out_specs=pl.BlockSpec(memory_space=pl.ANY),
        out_shape=jax.ShapeDtypeStruct((_ROWS, _COLS), jnp.float32),
        scratch_shapes=[
            pltpu.VMEM((_NCH, _CR, _COLS), jnp.float32),
            pltpu.SMEM((_NIDX + 1, 128), jnp.float32),
            pltpu.SemaphoreType.DMA,
            pltpu.SemaphoreType.DMA,
            pltpu.SemaphoreType.DMA,
        ],
    )(ti.astype(jnp.int32), weight, tv, f)
    return out
